# baseline (speedup 1.0000x reference)
import jax
import jax.numpy as jnp
from jax.experimental import pallas as pl
from jax.experimental.pallas import tpu as pltpu

B = 4
C = 4
N1 = 16384


def _copy_body(src_hbm, out_hbm, sem):
    copies = [
        pltpu.make_async_copy(src_hbm.at[b], out_hbm.at[b], sem)
        for b in range(B)
    ]
    for cp in copies:
        cp.start()
    for cp in copies:
        cp.wait()


def kernel(source, target, T_prev):
    del target, T_prev
    out = pl.pallas_call(
        _copy_body,
        out_shape=jax.ShapeDtypeStruct((B, C, N1), jnp.float32),
        in_specs=[pl.BlockSpec(memory_space=pltpu.MemorySpace.HBM)],
        out_specs=pl.BlockSpec(memory_space=pltpu.MemorySpace.HBM),
        scratch_shapes=[pltpu.SemaphoreType.DMA],
    )(source)
    return jnp.transpose(out, (0, 2, 1))


# speedup vs baseline: 3.6535x; 3.6535x over previous
import jax
import jax.numpy as jnp
from jax.experimental import pallas as pl
from jax.experimental.pallas import tpu as pltpu

B = 4
C = 4
N1 = 16384
SPLIT = 4


def _copy_body(src_ref, out_ref):
    out_ref[...] = src_ref[...]


def kernel(source, target, T_prev):
    del target, T_prev
    out = pl.pallas_call(
        _copy_body,
        out_shape=jax.ShapeDtypeStruct((B, C, N1), jnp.float32),
        grid=(B, SPLIT),
        in_specs=[pl.BlockSpec((1, C, N1 // SPLIT), lambda b, s: (b, 0, s))],
        out_specs=pl.BlockSpec((1, C, N1 // SPLIT), lambda b, s: (b, 0, s)),
    )(source)
    return jnp.transpose(out, (0, 2, 1))


# speedup vs baseline: 14.9389x; 4.0889x over previous
import jax
import jax.numpy as jnp
from jax.experimental import pallas as pl
from jax.experimental.pallas import tpu as pltpu

B = 4
C = 4
N1 = 16384
SPLIT2 = 2


def _copy_body(src_ref, out_ref):
    out_ref[...] = src_ref[...]


def kernel(source, target, T_prev):
    del target, T_prev
    out = pl.pallas_call(
        _copy_body,
        out_shape=jax.ShapeDtypeStruct((B, C, N1), jnp.float32),
        grid=(SPLIT2,),
        in_specs=[pl.BlockSpec((B // SPLIT2, C, N1), lambda i: (i, 0, 0))],
        out_specs=pl.BlockSpec((B // SPLIT2, C, N1), lambda i: (i, 0, 0)),
    )(source)
    return jnp.transpose(out, (0, 2, 1))
